# baseline (speedup 1.0000x reference)
import functools

import jax
import jax.numpy as jnp
from jax import lax
from jax.experimental import pallas as pl
from jax.experimental.pallas import tpu as pltpu
from jax.experimental.pallas import tpu_sc as plsc

D_MODEL = 64
SCALE = 8.0
IBLK = 128
A_BLK = 256


@functools.partial(jax.jit, static_argnames=("vocab",))
def _build_table(lut_t, vocab):
    info = plsc.get_sparse_core_info()
    nc, ns = info.num_cores, info.num_subcores
    nw = nc * ns
    nblk = vocab // A_BLK
    tail = vocab - nblk * A_BLK
    mesh = plsc.VectorSubcoreMesh(core_axis_name="c", subcore_axis_name="s")

    @functools.partial(
        pl.kernel,
        mesh=mesh,
        out_type=jax.ShapeDtypeStruct((vocab * D_MODEL,), jnp.float32),
        scratch_types=[
            pltpu.VMEM((2 * D_MODEL, A_BLK), jnp.float32),
            pltpu.VMEM((2 * A_BLK * D_MODEL,), jnp.float32),
            pltpu.SemaphoreType.DMA((2,)),
            pltpu.SemaphoreType.DMA((2,)),
        ],
        compiler_params=pltpu.CompilerParams(
            needs_layout_passes=False, disable_bounds_checks=True
        ),
    )
    def ka(lutt_hbm, tbl_hbm, tin, tout, sem_i, sem_o):
        wid = lax.axis_index("s") * nc + lax.axis_index("c")
        iota16 = lax.iota(jnp.int32, 16)

        def blk(k):
            return (k * nw + wid) * A_BLK

        def start_in(k, buf):
            pltpu.async_copy(
                lutt_hbm.at[:, pl.ds(blk(k), A_BLK)],
                tin.at[pl.ds(buf * D_MODEL, D_MODEL), :],
                sem_i.at[buf],
            )

        def wait_in(k, buf):
            pltpu.make_async_copy(
                lutt_hbm.at[:, pl.ds(blk(k), A_BLK)],
                tin.at[pl.ds(buf * D_MODEL, D_MODEL), :],
                sem_i.at[buf],
            ).wait()

        def start_out(k, buf):
            pltpu.async_copy(
                tout.at[pl.ds(buf * A_BLK * D_MODEL, A_BLK * D_MODEL)],
                tbl_hbm.at[pl.ds(blk(k) * D_MODEL, A_BLK * D_MODEL)],
                sem_o.at[buf],
            )

        def wait_out(k, buf):
            pltpu.make_async_copy(
                tout.at[pl.ds(buf * A_BLK * D_MODEL, A_BLK * D_MODEL)],
                tbl_hbm.at[pl.ds(blk(k) * D_MODEL, A_BLK * D_MODEL)],
                sem_o.at[buf],
            ).wait()

        def transpose_slab(buf):
            obase = buf * A_BLK * D_MODEL

            @plsc.parallel_loop(0, A_BLK, unroll=8)
            def _ubody(u):
                col16 = u + 0 * iota16
                for g4 in range(D_MODEL // 16):
                    rows16 = (buf * D_MODEL + g4 * 16) + iota16
                    v = plsc.load_gather(tin, [rows16, col16])
                    tout[pl.ds(obase + u * D_MODEL + g4 * 16, 16)] = v * SCALE

        nmine = (nblk - wid + nw - 1) // nw

        @pl.when(nmine > 0)
        def _():
            start_in(0, 0)

            def body(k, _):
                buf = k % 2

                @pl.when(k + 1 < nmine)
                def _():
                    start_in(k + 1, (k + 1) % 2)

                wait_in(k, buf)

                @pl.when(k >= 2)
                def _():
                    wait_out(k - 2, buf)

                transpose_slab(buf)
                start_out(k, buf)
                return 0

            lax.fori_loop(0, nmine, body, 0)

            for j in range(2):
                @pl.when(nmine >= j + 1)
                def _():
                    wait_out(nmine - 1 - j, (nmine - 1 - j) % 2)

        if tail:
            @pl.when(wid == 0)
            def _():
                i0 = nblk * A_BLK
                for dd in range(D_MODEL):
                    pltpu.async_copy(
                        lutt_hbm.at[dd, pl.ds(i0, tail)],
                        tin.at[dd, pl.ds(0, tail)],
                        sem_i.at[0],
                    )
                for dd in range(D_MODEL):
                    pltpu.make_async_copy(
                        lutt_hbm.at[dd, pl.ds(i0, tail)],
                        tin.at[dd, pl.ds(0, tail)],
                        sem_i.at[0],
                    ).wait()

                @plsc.parallel_loop(0, tail, unroll=8)
                def _tbody(u):
                    col16 = u + 0 * iota16
                    for g4 in range(D_MODEL // 16):
                        rows16 = g4 * 16 + iota16
                        v = plsc.load_gather(tin, [rows16, col16])
                        tout[pl.ds(u * D_MODEL + g4 * 16, 16)] = v * SCALE

                pltpu.sync_copy(
                    tout.at[pl.ds(0, tail * D_MODEL)],
                    tbl_hbm.at[pl.ds(i0 * D_MODEL, tail * D_MODEL)],
                )

    return ka(lut_t)


@functools.partial(jax.jit, static_argnames=("n_s", "n_b", "vocab"))
def _emb_gather(x_t, tbl1d, n_s, n_b, vocab):
    info = plsc.get_sparse_core_info()
    nc, ns = info.num_cores, info.num_subcores
    nw = nc * ns
    bpw = n_b // nw
    ndg = D_MODEL // 8
    nbt = n_b // IBLK
    tbl = tbl1d.reshape(vocab, D_MODEL)
    mesh = plsc.VectorSubcoreMesh(core_axis_name="c", subcore_axis_name="s")

    @functools.partial(
        pl.kernel,
        mesh=mesh,
        out_type=jax.ShapeDtypeStruct((n_s, ndg, nbt, 8, IBLK), jnp.float32),
        scratch_types=[
            pltpu.VMEM((n_s, bpw), jnp.int32),
            pltpu.VMEM((2 * 4 * bpw,), jnp.int32),
            pltpu.VMEM((2 * 4 * bpw, D_MODEL), jnp.float32),
            pltpu.VMEM((4 * ndg, 8, IBLK), jnp.float32),
            pltpu.SemaphoreType.DMA((2,)),
            pltpu.SemaphoreType.DMA((4,)),
        ],
        compiler_params=pltpu.CompilerParams(
            use_tc_tiling_on_sc=False,
            needs_layout_passes=False,
            disable_bounds_checks=True,
        ),
    )
    def kb(xt_hbm, tbl_hbm, out_hbm, xv, pid_v, rows_v, out_v, sem_g, sem_w):
        wid = lax.axis_index("s") * nc + lax.axis_index("c")
        b0 = wid * bpw
        pltpu.sync_copy(xt_hbm.at[:, pl.ds(b0, bpw)], xv)
        iota16 = lax.iota(jnp.int32, 16)

        def fill_pid(c, buf):
            for r in range(4):
                for g in range(bpw // 16):
                    pid_v[pl.ds(buf * 4 * bpw + r * bpw + g * 16, 16)] = xv[
                        4 * c + r, pl.ds(g * 16, 16)
                    ]

        def start_gather(c, buf):
            pltpu.async_copy(
                tbl_hbm.at[pid_v.at[pl.ds(buf * 4 * bpw, 4 * bpw)]],
                rows_v.at[pl.ds(buf * 4 * bpw, 4 * bpw), :],
                sem_g.at[buf],
            )

        def wait_gather(c, buf):
            pltpu.make_async_copy(
                tbl_hbm.at[pid_v.at[pl.ds(buf * 4 * bpw, 4 * bpw)]],
                rows_v.at[pl.ds(buf * 4 * bpw, 4 * bpw), :],
                sem_g.at[buf],
            ).wait()

        def start_write(s, buf):
            pltpu.async_copy(
                out_v.at[pl.ds(buf * ndg, ndg), :, :],
                out_hbm.at[s, :, wid, :, :],
                sem_w.at[buf],
            )

        def wait_write(s, buf):
            pltpu.make_async_copy(
                out_v.at[pl.ds(buf * ndg, ndg), :, :],
                out_hbm.at[s, :, wid, :, :],
                sem_w.at[buf],
            ).wait()

        def assemble(rowbase, obuf):
            for g in range(bpw // 16):
                rows16 = (rowbase + g * 16) + iota16

                @plsc.parallel_loop(0, D_MODEL, unroll=8)
                def _dbody(d):
                    col16 = d + 0 * iota16
                    v = plsc.load_gather(rows_v, [rows16, col16])
                    out_v[obuf * ndg + d // 8, d % 8, pl.ds(g * 16, 16)] = v

        n_c = n_s // 4
        fill_pid(0, 0)
        start_gather(0, 0)

        def cbody(c, _):
            buf = c % 2

            @pl.when(c + 1 < n_c)
            def _():
                fill_pid(c + 1, (c + 1) % 2)
                start_gather(c + 1, (c + 1) % 2)

            wait_gather(c, buf)

            for j in range(4):
                s = 4 * c + j

                @pl.when(s >= 4)
                def _():
                    wait_write(s - 4, j)

                assemble(buf * 4 * bpw + j * bpw, j)
                start_write(s, j)
            return 0

        lax.fori_loop(0, n_c, cbody, 0)
        for j in range(4):
            wait_write(n_s - 4 + j, j)

    return kb(x_t, tbl)


def kernel(x, lut):
    b, s = x.shape
    v, d = lut.shape
    x_t = jnp.transpose(x).astype(jnp.int32)
    lut_t = jnp.transpose(lut)
    tbl1d = _build_table(lut_t, v)
    out5 = _emb_gather(x_t, tbl1d, s, b, v)
    return out5.transpose(2, 4, 0, 1, 3).reshape(b, s, d)


# speedup vs baseline: 1.1755x; 1.1755x over previous
import functools
import math

import jax
import jax.numpy as jnp
from jax import lax
from jax.experimental import pallas as pl
from jax.experimental.pallas import tpu as pltpu
from jax.experimental.pallas import tpu_sc as plsc

D_MODEL = 64
SCALE = math.sqrt(D_MODEL)


@functools.partial(jax.jit, static_argnames=("n_s", "n_b"))
def _emb_lookup(x_t, lut_pairs, n_s, n_b):
    info = plsc.get_sparse_core_info()
    nc, ns = info.num_cores, info.num_subcores
    nw = nc * ns
    bpw = n_b // nw
    mesh = plsc.VectorSubcoreMesh(core_axis_name="c", subcore_axis_name="s")

    @functools.partial(
        pl.kernel,
        mesh=mesh,
        out_type=jax.ShapeDtypeStruct((n_s, D_MODEL, n_b), jnp.float32),
        scratch_types=[
            pltpu.VMEM((n_s, bpw), jnp.int32),
            pltpu.VMEM((2, bpw), jnp.int32),
            pltpu.VMEM((2 * bpw, 128), jnp.float32),
            pltpu.VMEM((2 * D_MODEL, bpw), jnp.float32),
            pltpu.SemaphoreType.DMA((2,)),
            pltpu.SemaphoreType.DMA((2,)),
        ],
        compiler_params=pltpu.CompilerParams(
            needs_layout_passes=False, disable_bounds_checks=True
        ),
    )
    def k(xt_hbm, tbl_hbm, out_hbm, xv, pid_v, rows_v, out_v, sem_g, sem_w):
        wid = lax.axis_index("s") * nc + lax.axis_index("c")
        b0 = wid * bpw
        pltpu.sync_copy(xt_hbm.at[:, pl.ds(b0, bpw)], xv)
        iota16 = lax.iota(jnp.int32, 16)

        def compute_pid(s, buf):
            for g in range(bpw // 16):
                idx16 = xv[s, pl.ds(g * 16, 16)]
                pid_v[buf, pl.ds(g * 16, 16)] = idx16 >> 1

        def start_gather(buf):
            pltpu.async_copy(
                tbl_hbm.at[pid_v.at[buf]],
                rows_v.at[pl.ds(buf * bpw, bpw), :],
                sem_g.at[buf],
            )

        def wait_gather(buf):
            pltpu.make_async_copy(
                tbl_hbm.at[pid_v.at[buf]],
                rows_v.at[pl.ds(buf * bpw, bpw), :],
                sem_g.at[buf],
            ).wait()

        def start_write(s, buf):
            pltpu.async_copy(
                out_v.at[pl.ds(buf * D_MODEL, D_MODEL), :],
                out_hbm.at[s, :, pl.ds(b0, bpw)],
                sem_w.at[buf],
            )

        def wait_write(s, buf):
            pltpu.make_async_copy(
                out_v.at[pl.ds(buf * D_MODEL, D_MODEL), :],
                out_hbm.at[s, :, pl.ds(b0, bpw)],
                sem_w.at[buf],
            ).wait()

        def assemble(s, buf):
            for g in range(bpw // 16):
                idx16 = xv[s, pl.ds(g * 16, 16)]
                half = (idx16 & 1) << 6
                rows16 = (buf * bpw + g * 16) + iota16

                @plsc.parallel_loop(0, D_MODEL, unroll=8)
                def _dbody(d):
                    col16 = half + d
                    v = plsc.load_gather(rows_v, [rows16, col16])
                    out_v[buf * D_MODEL + d, pl.ds(g * 16, 16)] = v * SCALE

        compute_pid(0, 0)
        start_gather(0)

        def sbody(s, _):
            buf = s % 2
            nbuf = (s + 1) % 2

            @pl.when(s < n_s - 1)
            def _():
                compute_pid(s + 1, nbuf)
                start_gather(nbuf)

            wait_gather(buf)

            @pl.when(s >= 2)
            def _():
                wait_write(s - 2, buf)

            assemble(s, buf)
            start_write(s, buf)
            return 0

        lax.fori_loop(0, n_s, sbody, 0)
        wait_write(n_s - 2, 0)
        wait_write(n_s - 1, 1)

    return k(x_t, lut_pairs)


def kernel(x, lut):
    b, s = x.shape
    v, d = lut.shape
    x_t = jnp.transpose(x).astype(jnp.int32)
    lut_pairs = lut.reshape(v // 2, 2 * d)
    out3 = _emb_lookup(x_t, lut_pairs, s, b)
    return jnp.transpose(out3, (2, 0, 1))
